# Initial kernel scaffold; baseline (speedup 1.0000x reference)
import jax
import jax.numpy as jnp
from jax.experimental import pallas as pl


def kernel(hidden_states, target_ids, Wr, br, col_weight, col_bias):
    raise NotImplementedError("write your pallas kernel here")


# speedup vs baseline: 2.5245x; 2.5245x over previous
import functools

import jax
import jax.numpy as jnp
from jax.experimental import pallas as pl
from jax.experimental.pallas import tpu as pltpu

_R = 64


def _loss_kernel(hs_ref, rowids_ref, colids_ref, Wr_ref, br_ref, cw_ref,
                 cb_ref, out_ref, logits_acc):
    g = pl.program_id(0)
    n_experts = pl.num_programs(0)

    p = jnp.dot(hs_ref[...], cw_ref[0], preferred_element_type=jnp.float32)
    p = p + cb_ref[0]

    mask = rowids_ref[...] == g
    logits_acc[...] = jnp.where(mask, p, logits_acc[...])

    @pl.when(g == n_experts - 1)
    def _finalize():
        n = hs_ref.shape[0]
        col_logits = logits_acc[...]
        row_logits = (
            jnp.dot(hs_ref[...], Wr_ref[...],
                    preferred_element_type=jnp.float32) + br_ref[...])

        def mean_nll(logits, labels):
            m = jnp.max(logits, axis=-1, keepdims=True)
            lse = m + jnp.log(
                jnp.sum(jnp.exp(logits - m), axis=-1, keepdims=True))
            lane = jax.lax.broadcasted_iota(jnp.int32, logits.shape, 1)
            sel = jnp.sum(
                jnp.where(lane == labels, logits, 0.0), axis=-1,
                keepdims=True)
            return jnp.sum(lse - sel, axis=0, keepdims=True) / n

        loss = (mean_nll(row_logits, rowids_ref[...])
                + mean_nll(col_logits, colids_ref[...]))
        out_ref[...] = loss


@functools.partial(jax.jit, static_argnames=())
def kernel(hidden_states, target_ids, Wr, br, col_weight, col_bias):
    d = hidden_states.shape[-1]
    r = br.shape[0]
    hs = hidden_states.reshape(-1, d)
    n = hs.shape[0]
    ids = target_ids.reshape(-1).astype(jnp.int32)
    row_ids = (ids // r).reshape(n, 1)
    col_ids = (ids % r).reshape(n, 1)

    hs_bf = hs.astype(jnp.bfloat16)
    cw_bf = col_weight.astype(jnp.bfloat16)
    wr_bf = Wr.astype(jnp.bfloat16)

    out = pl.pallas_call(
        _loss_kernel,
        grid=(r,),
        in_specs=[
            pl.BlockSpec((n, d), lambda g: (0, 0)),
            pl.BlockSpec((n, 1), lambda g: (0, 0)),
            pl.BlockSpec((n, 1), lambda g: (0, 0)),
            pl.BlockSpec((d, r), lambda g: (0, 0)),
            pl.BlockSpec((1, r), lambda g: (0, 0)),
            pl.BlockSpec((1, d, r), lambda g: (g, 0, 0)),
            pl.BlockSpec((1, 1, r), lambda g: (g, 0, 0)),
        ],
        out_specs=pl.BlockSpec((1, 1), lambda g: (0, 0)),
        out_shape=jax.ShapeDtypeStruct((1, 1), jnp.float32),
        scratch_shapes=[pltpu.VMEM((n, r), jnp.float32)],
        compiler_params=pltpu.CompilerParams(
            dimension_semantics=("arbitrary",)),
    )(hs_bf, row_ids, col_ids, wr_bf, br.reshape(1, r), cw_bf,
      col_bias.reshape(r, 1, r))
    return out[0, 0]


# speedup vs baseline: 4.7933x; 1.8987x over previous
import functools

import jax
import jax.numpy as jnp
from jax.experimental import pallas as pl
from jax.experimental.pallas import tpu as pltpu

_NEG = -1e30


def _loss_kernel(hs_ref, rowids_ref, colids_ref, Wr_ref, br_ref, wcat_ref,
                 cb_ref, out_ref, *, n_total, r):
    i = pl.program_id(0)

    rows = rowids_ref[...]
    cols = colids_ref[...]
    hs = hs_ref[...]

    p = jnp.dot(hs, wcat_ref[...], preferred_element_type=jnp.float32)
    p = p + cb_ref[...]

    lane = jax.lax.broadcasted_iota(jnp.int32, p.shape, 1)
    in_slice = (lane // r) == rows
    masked = jnp.where(in_slice, p, _NEG)
    m = jnp.max(masked, axis=-1, keepdims=True)
    s = jnp.sum(jnp.exp(masked - m), axis=-1, keepdims=True)
    lse = m + jnp.log(s)
    sel = jnp.sum(jnp.where(lane == rows * r + cols, p, 0.0),
                  axis=-1, keepdims=True)
    nll_col = jnp.sum(lse - sel, axis=0, keepdims=True)

    q = jnp.dot(hs, Wr_ref[...], preferred_element_type=jnp.float32)
    q = q + br_ref[...]
    lane_r = jax.lax.broadcasted_iota(jnp.int32, q.shape, 1)
    mq = jnp.max(q, axis=-1, keepdims=True)
    sq = jnp.sum(jnp.exp(q - mq), axis=-1, keepdims=True)
    lse_q = mq + jnp.log(sq)
    sel_q = jnp.sum(jnp.where(lane_r == rows, q, 0.0), axis=-1, keepdims=True)
    nll_row = jnp.sum(lse_q - sel_q, axis=0, keepdims=True)

    partial = (nll_col + nll_row) / n_total

    @pl.when(i == 0)
    def _init():
        out_ref[...] = jnp.zeros_like(out_ref)

    out_ref[...] += partial


@jax.jit
def kernel(hidden_states, target_ids, Wr, br, col_weight, col_bias):
    d = hidden_states.shape[-1]
    r = br.shape[0]
    hs = hidden_states.reshape(-1, d)
    n = hs.shape[0]
    ids = target_ids.reshape(-1).astype(jnp.int32)
    row_ids = (ids // r).reshape(n, 1)
    col_ids = (ids % r).reshape(n, 1)

    hs_bf = hs.astype(jnp.bfloat16)
    wcat_bf = col_weight.transpose(1, 0, 2).reshape(d, r * r).astype(
        jnp.bfloat16)
    wr_bf = Wr.astype(jnp.bfloat16)
    cb_flat = col_bias.reshape(1, r * r)

    tb = 512
    grid = (n // tb,)

    out = pl.pallas_call(
        functools.partial(_loss_kernel, n_total=n, r=r),
        grid=grid,
        in_specs=[
            pl.BlockSpec((tb, d), lambda i: (i, 0)),
            pl.BlockSpec((tb, 1), lambda i: (i, 0)),
            pl.BlockSpec((tb, 1), lambda i: (i, 0)),
            pl.BlockSpec((d, r), lambda i: (0, 0)),
            pl.BlockSpec((1, r), lambda i: (0, 0)),
            pl.BlockSpec((d, r * r), lambda i: (0, 0)),
            pl.BlockSpec((1, r * r), lambda i: (0, 0)),
        ],
        out_specs=pl.BlockSpec((1, 1), lambda i: (0, 0)),
        out_shape=jax.ShapeDtypeStruct((1, 1), jnp.float32),
        compiler_params=pltpu.CompilerParams(
            dimension_semantics=("arbitrary",)),
    )(hs_bf, row_ids, col_ids, wr_bf, br.reshape(1, r), wcat_bf, cb_flat)
    return out[0, 0]
